# baseline (speedup 1.0000x reference)
import functools
import jax
import jax.numpy as jnp
from jax import lax
from jax.experimental import pallas as pl
from jax.experimental.pallas import tpu as pltpu
from jax.experimental.pallas import tpu_sc as plsc

_NW = 32
_CHUNK = 16384
_UNROLL = 16


def _make_sc(total):
    per_w = total // _NW
    n_chunks = per_w // _CHUNK
    mesh = plsc.VectorSubcoreMesh(core_axis_name="c", subcore_axis_name="s")

    @functools.partial(
        pl.kernel,
        mesh=mesh,
        out_type=jax.ShapeDtypeStruct((total,), jnp.float32),
        scratch_types=[
            pltpu.VMEM((_CHUNK,), jnp.float32),
            pltpu.VMEM((_CHUNK,), jnp.float32),
            pltpu.VMEM((_CHUNK,), jnp.float32),
            pltpu.VMEM((_CHUNK,), jnp.float32),
            pltpu.SemaphoreType.DMA,
            pltpu.SemaphoreType.DMA,
            pltpu.SemaphoreType.DMA,
            pltpu.SemaphoreType.DMA,
        ],
    )
    def k(x_hbm, out_hbm, in0, in1, out0, out1, si0, si1, so0, so1):
        wid = lax.axis_index("s") * 2 + lax.axis_index("c")
        base = wid * per_w
        lane = lax.iota(jnp.int32, 16)
        grp = lane & ~3
        p1 = (grp | ((lane + 1) & 3))[:, None]
        p2 = (grp | ((lane + 2) & 3))[:, None]
        dnums = lax.GatherDimensionNumbers(
            offset_dims=(), collapsed_slice_dims=(0,), start_index_map=(0,)
        )

        def _perm(v, p):
            return lax.gather(
                v, p, dnums, slice_sizes=(1,),
                mode=lax.GatherScatterMode.PROMISE_IN_BOUNDS,
            )

        def _compute(buf_in, buf_out):
            @plsc.parallel_loop(0, _CHUNK, step=16, unroll=_UNROLL)
            def vec_body(b):
                v = buf_in[pl.ds(b, 16)]
                y1 = _perm(v, p1)
                mx1 = jnp.maximum(v, y1)
                mn1 = jnp.minimum(v, y1)
                mx2 = _perm(mx1, p2)
                mn2 = _perm(mn1, p2)
                second = jnp.maximum(
                    jnp.minimum(mx1, mx2), jnp.maximum(mn1, mn2)
                )
                buf_out[pl.ds(b, 16)] = jnp.where(v >= second, v, 0.0)

        def pair_body(i, carry):
            offa = base + (2 * i) * _CHUNK
            offb = offa + _CHUNK
            ha = pltpu.async_copy(x_hbm.at[pl.ds(offa, _CHUNK)], in0, si0)
            hb = pltpu.async_copy(x_hbm.at[pl.ds(offb, _CHUNK)], in1, si1)
            ha.wait()
            _compute(in0, out0)
            hoa = pltpu.async_copy(out0, out_hbm.at[pl.ds(offa, _CHUNK)], so0)
            hb.wait()
            _compute(in1, out1)
            hob = pltpu.async_copy(out1, out_hbm.at[pl.ds(offb, _CHUNK)], so1)
            hoa.wait()
            hob.wait()
            return carry

        lax.fori_loop(0, n_chunks // 2, pair_body, 0)

    return k


def kernel(input):
    n, d = input.shape
    flat = input.reshape(n * d)
    out = _make_sc(n * d)(flat)
    return out.reshape(n, d)


# speedup vs baseline: 1.2022x; 1.2022x over previous
import functools
import jax
import jax.numpy as jnp
from jax import lax
from jax.experimental import pallas as pl
from jax.experimental.pallas import tpu as pltpu
from jax.experimental.pallas import tpu_sc as plsc

_NW = 32
_CHUNK = 16384
_UNROLL = 8


def _make_sc(total):
    per_w = total // _NW
    n_chunks = per_w // _CHUNK
    n_iters = n_chunks // 4
    mesh = plsc.VectorSubcoreMesh(core_axis_name="c", subcore_axis_name="s")

    @functools.partial(
        pl.kernel,
        mesh=mesh,
        out_type=jax.ShapeDtypeStruct((total,), jnp.float32),
        scratch_types=[
            pltpu.VMEM((_CHUNK,), jnp.float32),
            pltpu.VMEM((_CHUNK,), jnp.float32),
            pltpu.VMEM((_CHUNK,), jnp.float32),
            pltpu.VMEM((_CHUNK,), jnp.float32),
            pltpu.VMEM((_CHUNK,), jnp.float32),
            pltpu.VMEM((_CHUNK,), jnp.float32),
            pltpu.SemaphoreType.DMA,
            pltpu.SemaphoreType.DMA,
            pltpu.SemaphoreType.DMA,
            pltpu.SemaphoreType.DMA,
            pltpu.SemaphoreType.DMA,
            pltpu.SemaphoreType.DMA,
        ],
    )
    def k(x_hbm, out_hbm, in0, in1, in2, in3, out0, out1,
          si0, si1, si2, si3, so0, so1):
        wid = lax.axis_index("s") * 2 + lax.axis_index("c")
        base = wid * per_w
        lane = lax.iota(jnp.int32, 16)
        grp = lane & ~3
        p1 = (grp | ((lane + 1) & 3))[:, None]
        p2 = (grp | ((lane + 2) & 3))[:, None]
        dnums = lax.GatherDimensionNumbers(
            offset_dims=(), collapsed_slice_dims=(0,), start_index_map=(0,)
        )

        def _perm(v, p):
            return lax.gather(
                v, p, dnums, slice_sizes=(1,),
                mode=lax.GatherScatterMode.PROMISE_IN_BOUNDS,
            )

        def _compute(buf_in, buf_out):
            @plsc.parallel_loop(0, _CHUNK, step=16, unroll=_UNROLL)
            def vec_body(b):
                v = buf_in[pl.ds(b, 16)]
                y1 = _perm(v, p1)
                mx1 = jnp.maximum(v, y1)
                mn1 = jnp.minimum(v, y1)
                mx2 = _perm(mx1, p2)
                mn2 = _perm(mn1, p2)
                second = jnp.maximum(
                    jnp.minimum(mx1, mx2), jnp.maximum(mn1, mn2)
                )
                buf_out[pl.ds(b, 16)] = jnp.where(v >= second, v, 0.0)

        ins = ((in0, si0), (in1, si1), (in2, si2), (in3, si3))
        outs = ((out0, so0), (out1, so1))

        for q in range(4):
            pltpu.async_copy(
                x_hbm.at[pl.ds(base + q * _CHUNK, _CHUNK)], ins[q][0],
                ins[q][1],
            )

        def iter_body(i, carry):
            c0 = 4 * i
            for q in range(4):
                ib, isem = ins[q]
                ob, osem = outs[q % 2]
                off = base + (c0 + q) * _CHUNK
                pltpu.make_async_copy(
                    x_hbm.at[pl.ds(off, _CHUNK)], ib, isem
                ).wait()

                prev_off = base + lax.rem(
                    c0 + q - 2 + n_chunks, n_chunks
                ) * _CHUNK

                @pl.when(jnp.logical_or(i > 0, q >= 2))
                def _():
                    pltpu.make_async_copy(
                        ob, out_hbm.at[pl.ds(prev_off, _CHUNK)], osem
                    ).wait()

                _compute(ib, ob)
                pltpu.async_copy(ob, out_hbm.at[pl.ds(off, _CHUNK)], osem)
                nxt = base + lax.rem(c0 + q + 4, n_chunks) * _CHUNK
                pltpu.async_copy(x_hbm.at[pl.ds(nxt, _CHUNK)], ib, isem)
            return carry

        lax.fori_loop(0, n_iters, iter_body, 0)

        for q in range(4):
            pltpu.make_async_copy(
                x_hbm.at[pl.ds(base + q * _CHUNK, _CHUNK)], ins[q][0],
                ins[q][1],
            ).wait()
        for q in range(2):
            pltpu.make_async_copy(
                outs[q][0],
                out_hbm.at[pl.ds(base + q * _CHUNK, _CHUNK)], outs[q][1],
            ).wait()

    return k


def kernel(input):
    n, d = input.shape
    flat = input.reshape(n * d)
    out = _make_sc(n * d)(flat)
    return out.reshape(n, d)
